# baseline (speedup 1.0000x reference)
import jax
import jax.numpy as jnp
from jax import lax
from jax.experimental import pallas as pl
from jax.experimental.pallas import tpu as pltpu

_BC = 65536
_IMAX = 2**31 - 1
_N = 1000000


def _scan_body(zm2_ref, blkT_ref, out_ref, bestv_ref, besti_ref):
    i = pl.program_id(0)
    nb = pl.num_programs(0)

    @pl.when(i == 0)
    def _init():
        bestv_ref[...] = jnp.full(bestv_ref.shape, jnp.inf, jnp.float32)
        besti_ref[...] = jnp.zeros(besti_ref.shape, jnp.int32)

    blkT = blkT_ref[...]
    zm2 = zm2_ref[...]
    prod = lax.dot_general(
        zm2, blkT, (((1,), (0,)), ((), ())),
        preferred_element_type=jnp.float32)
    ksq = jnp.sum(blkT * blkT, axis=0, keepdims=True)
    lane1 = lax.broadcasted_iota(jnp.int32, (1, _BC), 1)
    ksq = jnp.where(lane1 < (_N - i * _BC), ksq, jnp.inf)
    metric = prod + ksq

    minv = jnp.min(metric, axis=1, keepdims=True)
    lanes = lax.broadcasted_iota(jnp.int32, metric.shape, 1)
    cand = jnp.where(metric == minv, lanes, _IMAX)
    minl = jnp.min(cand, axis=1, keepdims=True)

    prevv = bestv_ref[...]
    previ = besti_ref[...]
    upd = minv < prevv
    bestv_ref[...] = jnp.where(upd, minv, prevv)
    newi = jnp.where(upd, minl + i * _BC, previ)
    besti_ref[...] = newi

    @pl.when(i == nb - 1)
    def _fin():
        out_ref[...] = jnp.broadcast_to(newi.reshape(1, 32), out_ref.shape)


def _gather_body(idx_ref, *refs):
    out_ref = refs[-1]
    lane128 = lax.broadcasted_iota(jnp.int32, (16, 128), 1)
    cols = []
    for q in range(32):
        p = idx_ref[q] % 128
        cols.append(jnp.sum(jnp.where(lane128 == p, refs[q][...], 0.0),
                            axis=1, keepdims=True))
    out_ref[...] = jnp.concatenate(cols, axis=1)


def kernel(z, all_z):
    b, l, d = z.shape
    zf = jnp.reshape(z, (-1, d))
    zm2 = -2.0 * zf
    all_zT = all_z.T
    n = all_z.shape[0]
    nb = (n + _BC - 1) // _BC
    idx8 = pl.pallas_call(
        _scan_body,
        grid=(nb,),
        in_specs=[
            pl.BlockSpec((32, 16), lambda i: (0, 0)),
            pl.BlockSpec((16, _BC), lambda i: (0, i)),
        ],
        out_specs=pl.BlockSpec((8, 32), lambda i: (0, 0)),
        out_shape=jax.ShapeDtypeStruct((8, 32), jnp.int32),
        scratch_shapes=[
            pltpu.VMEM((32, 1), jnp.float32),
            pltpu.VMEM((32, 1), jnp.int32),
        ],
    )(zm2, all_zT)
    idx = idx8[0]
    def _mk_spec(q):
        return pl.BlockSpec((16, 128), lambda i, idx_ref, q=q: (0, idx_ref[q] // 128))

    bvec = pl.pallas_call(
        _gather_body,
        grid_spec=pltpu.PrefetchScalarGridSpec(
            num_scalar_prefetch=1,
            grid=(1,),
            in_specs=[_mk_spec(q) for q in range(32)],
            out_specs=pl.BlockSpec((16, 32), lambda i, idx_ref: (0, 0)),
        ),
        out_shape=jax.ShapeDtypeStruct((16, 32), jnp.float32),
    )(idx, *([all_zT] * 32))
    return jnp.reshape(bvec.T, (b, l, d))


# speedup vs baseline: 1.4880x; 1.4880x over previous
import jax
import jax.numpy as jnp
from jax import lax
from jax.experimental import pallas as pl
from jax.experimental.pallas import tpu as pltpu

_BC = 65536
_IMAX = 2**31 - 1
_N = 1000000


def _scan_body(zaug_ref, blkT_ref, out_ref, bestv_ref, besti_ref):
    i = pl.program_id(0)
    nb = pl.num_programs(0)

    @pl.when(i == 0)
    def _init():
        bestv_ref[...] = jnp.full(bestv_ref.shape, jnp.inf, jnp.float32)
        besti_ref[...] = jnp.zeros(besti_ref.shape, jnp.int32)

    blkT = blkT_ref[...]
    zaug = zaug_ref[...]
    lane1 = lax.broadcasted_iota(jnp.int32, (1, _BC), 1)
    ok = lane1 < (_N - i * _BC)
    aug = jnp.concatenate(
        [blkT, jnp.where(ok, blkT * blkT, 1e30)], axis=0)
    metric = lax.dot_general(
        zaug, aug, (((1,), (0,)), ((), ())),
        preferred_element_type=jnp.float32)
    _C = 512
    R = jnp.full((32, _C), jnp.inf, jnp.float32)
    I = jnp.zeros((32, _C), jnp.int32)
    for c in range(_BC // _C):
        m = lax.slice(metric, (0, c * _C), (32, (c + 1) * _C))
        better = m < R
        R = jnp.where(better, m, R)
        I = jnp.where(better, c, I)

    minv = jnp.min(R, axis=1, keepdims=True)
    slots = lax.broadcasted_iota(jnp.int32, (32, _C), 1)
    idxs = I * _C + slots
    cand = jnp.where(R == minv, idxs, _IMAX)
    minl = jnp.min(cand, axis=1, keepdims=True)

    prevv = bestv_ref[...]
    previ = besti_ref[...]
    upd = minv < prevv
    bestv_ref[...] = jnp.where(upd, minv, prevv)
    newi = jnp.where(upd, minl + i * _BC, previ)
    besti_ref[...] = newi

    @pl.when(i == nb - 1)
    def _fin():
        out_ref[...] = jnp.broadcast_to(newi.reshape(1, 32), out_ref.shape)


def _gather_body(idx_ref, *refs):
    out_ref = refs[-1]
    lane128 = lax.broadcasted_iota(jnp.int32, (16, 128), 1)
    cols = []
    for q in range(32):
        p = idx_ref[q] % 128
        cols.append(jnp.sum(jnp.where(lane128 == p, refs[q][...], 0.0),
                            axis=1, keepdims=True))
    out_ref[...] = jnp.concatenate(cols, axis=1)


def kernel(z, all_z):
    b, l, d = z.shape
    zf = jnp.reshape(z, (-1, d))
    zaug = jnp.concatenate([-2.0 * zf, jnp.ones((b * l, d), jnp.float32)],
                           axis=1)
    all_zT = all_z.T
    n = all_z.shape[0]
    nb = (n + _BC - 1) // _BC
    idx8 = pl.pallas_call(
        _scan_body,
        grid=(nb,),
        in_specs=[
            pl.BlockSpec((32, 32), lambda i: (0, 0)),
            pl.BlockSpec((16, _BC), lambda i: (0, i)),
        ],
        out_specs=pl.BlockSpec((8, 32), lambda i: (0, 0)),
        out_shape=jax.ShapeDtypeStruct((8, 32), jnp.int32),
        scratch_shapes=[
            pltpu.VMEM((32, 1), jnp.float32),
            pltpu.VMEM((32, 1), jnp.int32),
        ],
    )(zaug, all_zT)
    idx = idx8[0]
    def _mk_spec(q):
        return pl.BlockSpec((16, 128), lambda i, idx_ref, q=q: (0, idx_ref[q] // 128))

    bvec = pl.pallas_call(
        _gather_body,
        grid_spec=pltpu.PrefetchScalarGridSpec(
            num_scalar_prefetch=1,
            grid=(1,),
            in_specs=[_mk_spec(q) for q in range(32)],
            out_specs=pl.BlockSpec((16, 32), lambda i, idx_ref: (0, 0)),
        ),
        out_shape=jax.ShapeDtypeStruct((16, 32), jnp.float32),
    )(idx, *([all_zT] * 32))
    return jnp.reshape(bvec.T, (b, l, d))
